# Initial kernel scaffold; baseline (speedup 1.0000x reference)
import jax
import jax.numpy as jnp
from jax.experimental import pallas as pl


def kernel(input, dim, index, src):
    raise NotImplementedError("write your pallas kernel here")


# speedup vs baseline: 31.0321x; 31.0321x over previous
import functools

import jax
import jax.numpy as jnp
from jax import lax
from jax.experimental import pallas as pl
from jax.experimental.pallas import tpu as pltpu
from jax.experimental.pallas import tpu_sc as plsc

_LANES = 16


@functools.lru_cache(maxsize=None)
def _make_scatter_kernel(M, D, B, chunk, unroll):
  mesh = plsc.VectorSubcoreMesh(core_axis_name="c", subcore_axis_name="s")
  nc, ns = mesh.num_cores, mesh.num_subcores
  nw = nc * ns
  cols_per_w = D // nw
  n_chunks = B // chunk
  n_vregs = chunk // _LANES

  @functools.partial(
      pl.kernel,
      out_type=jax.ShapeDtypeStruct((D, M), jnp.float32),
      mesh=mesh,
      scratch_types=[
          pltpu.VMEM((M,), jnp.float32),
          pltpu.VMEM((chunk,), jnp.int32),
          pltpu.VMEM((chunk,), jnp.float32),
      ],
      compiler_params=pltpu.CompilerParams(needs_layout_passes=False),
  )
  def scatter_kernel(inpT, idxT, srcT, outT, colbuf, idxbuf, srcbuf):
    wid = lax.axis_index("s") * nc + lax.axis_index("c")
    for c in range(cols_per_w):
      j = wid * cols_per_w + c
      pltpu.sync_copy(inpT.at[j], colbuf)
      for ch in range(n_chunks):
        pltpu.sync_copy(idxT.at[j, pl.ds(ch * chunk, chunk)], idxbuf)
        pltpu.sync_copy(srcT.at[j, pl.ds(ch * chunk, chunk)], srcbuf)

        def vbody(v, carry):
          base = v * _LANES
          idxv = idxbuf[pl.ds(base, _LANES)]
          srcv = srcbuf[pl.ds(base, _LANES)]
          _, keep = plsc.scan_count(idxv)
          plsc.store_scatter(colbuf, [idxv], srcv, mask=keep)
          return carry

        lax.fori_loop(0, n_vregs, vbody, 0, unroll=unroll)
      pltpu.sync_copy(colbuf, outT.at[j])

  return scatter_kernel


def kernel(input, dim, index, src):
  M, D = input.shape
  B = index.shape[0]
  idx = index + jnp.asarray(dim, index.dtype)
  f = _make_scatter_kernel(M, D, B, 4096, 8)
  outT = f(input.T, idx.T, src.T)
  return outT.T
